# baseline (device time: 37497 ns/iter reference)
import jax
import jax.numpy as jnp
from jax import lax
from jax.experimental import pallas as pl
from jax.experimental.pallas import tpu as pltpu

N_DEV = 4


def kernel(x, Win0, Wout0, Win1, Wout1, Win2, Wout2):
    M, D = x.shape
    H = Win0.shape[1]
    B = N_DEV * M
    M2 = M // 2

    def body(x_ref, win0_ref, wout0_ref, win1_ref, wout1_ref, win2_ref,
             wout2_ref, out_ref, X0, X1, X2, prbuf, sbuf, winbf, woutbf,
             send_sems, recv_sems):
        my = lax.axis_index("i")
        all_descs = []

        for l, (wi, wo) in enumerate(((win0_ref, wout0_ref),
                                      (win1_ref, wout1_ref),
                                      (win2_ref, wout2_ref))):
            winbf[l, :, :] = wi[:, :].astype(jnp.bfloat16)
            woutbf[l, :, :] = wo[:, :].astype(jnp.bfloat16)

        barrier_sem = pltpu.get_barrier_semaphore()
        for k in range(1, N_DEV):
            pl.semaphore_signal(
                barrier_sem, inc=1,
                device_id=((my + k) % N_DEV,),
                device_id_type=pl.DeviceIdType.MESH,
            )
        pl.semaphore_wait(barrier_sem, N_DEV - 1)

        def compute_half(xc, l):
            h = jnp.dot(xc, winbf[l, :, :],
                        preferred_element_type=jnp.float32)
            h = jnp.maximum(h, 0.0).astype(jnp.bfloat16)
            return jnp.dot(h, woutbf[l, :, :],
                           preferred_element_type=jnp.float32)

        def layer_step(l, Xl, get_half):
            xdescs = {}
            own_p = {}
            for half in (0, 1):
                xh = get_half(half).astype(jnp.bfloat16)
                rows = pl.ds(my * M + half * M2, M2)
                Xl[rows, :] = xh
                for k in range(1, N_DEV):
                    t = (my + k) % N_DEV
                    sl = (k - 1) * 2 + half
                    de = pltpu.make_async_remote_copy(
                        src_ref=Xl.at[rows, :],
                        dst_ref=Xl.at[rows, :],
                        send_sem=send_sems.at[2 * l, sl],
                        recv_sem=recv_sems.at[2 * l, sl],
                        device_id=(t,),
                        device_id_type=pl.DeviceIdType.MESH,
                    )
                    de.start()
                    xdescs[(k, half)] = de
                own_p[half] = compute_half(xh, l)
            all_descs.extend(xdescs.values())

            pdescs = {}
            for k, half in ((1, 0), (3, 0), (1, 1), (3, 1), (2, 0), (2, 1)):
                xdescs[(k, half)].wait_recv()
                s = (my - k) % N_DEV
                xk = Xl[pl.ds(s * M + half * M2, M2), :]
                pk = compute_half(xk, l)
                psl = (3 - k) * 2 + half
                sbuf[l, psl, :, :] = pk.astype(jnp.bfloat16)
                de = pltpu.make_async_remote_copy(
                    src_ref=sbuf.at[l, psl],
                    dst_ref=prbuf.at[l, psl],
                    send_sem=send_sems.at[2 * l + 1, psl],
                    recv_sem=recv_sems.at[2 * l + 1, psl],
                    device_id=(s,),
                    device_id_type=pl.DeviceIdType.MESH,
                )
                de.start()
                pdescs[(k, half)] = de
            all_descs.extend(pdescs.values())

            def next_get_half(half):
                tot = own_p[half]
                for k in (1, 3, 2):
                    pdescs[(k, half)].wait_recv()
                    tot = tot + prbuf[l, (3 - k) * 2 + half].astype(
                        jnp.float32
                    )
                return tot

            return next_get_half

        get_half = lambda half: x_ref[pl.ds(half * M2, M2), :]
        get_half = layer_step(0, X0, get_half)
        get_half = layer_step(1, X1, get_half)
        get_half = layer_step(2, X2, get_half)
        for half in (0, 1):
            out_ref[pl.ds(half * M2, M2), :] = get_half(half)
        for de in all_descs:
            de.wait_send()

    return pl.pallas_call(
        body,
        out_shape=jax.ShapeDtypeStruct((M, D), jnp.float32),
        in_specs=[pl.BlockSpec(memory_space=pltpu.VMEM)] * 7,
        out_specs=pl.BlockSpec(memory_space=pltpu.VMEM),
        scratch_shapes=[
            pltpu.VMEM((B, D), jnp.bfloat16),
            pltpu.VMEM((B, D), jnp.bfloat16),
            pltpu.VMEM((B, D), jnp.bfloat16),
            pltpu.VMEM((3, 6, M2, D), jnp.bfloat16),
            pltpu.VMEM((3, 6, M2, D), jnp.bfloat16),
            pltpu.VMEM((3, D, H), jnp.bfloat16),
            pltpu.VMEM((3, H, D), jnp.bfloat16),
            pltpu.SemaphoreType.DMA((6, 6)),
            pltpu.SemaphoreType.DMA((6, 6)),
        ],
        compiler_params=pltpu.CompilerParams(collective_id=0),
    )(x, Win0, Wout0, Win1, Wout1, Win2, Wout2)
